# baseline (speedup 1.0000x reference)
import jax
import jax.numpy as jnp
from jax.experimental import pallas as pl
from jax.experimental.pallas import tpu as pltpu

_HIDDEN = 4096
_EXPERTS = 64
_K = 8
_CHUNK = 128
_NSUB = 8
_BLOCK = _CHUNK * _NSUB
_NSLOT = 2 * _NSUB


def _topk_softmax(logits, tw_ref, te_ref, c):
    eidx = jax.lax.broadcasted_iota(jnp.int32, logits.shape, 0)
    l = logits
    vals = []
    idxs = []
    for _ in range(_K):
        m = jnp.max(l, axis=0, keepdims=True)
        idx = jnp.min(jnp.where(l == m, eidx, _EXPERTS),
                      axis=0, keepdims=True)
        vals.append(m)
        idxs.append(idx)
        l = jnp.where(eidx == idx, -jnp.inf, l)
    v = jnp.concatenate(vals, axis=0)
    e = jnp.exp(v - v[0:1])
    wts = e / jnp.sum(e, axis=0, keepdims=True)
    tw_ref[c * _CHUNK:(c + 1) * _CHUNK, :] = wts.T
    te_ref[c * _CHUNK:(c + 1) * _CHUNK, :] = jnp.concatenate(idxs, axis=0).T


def _router_step(x_hbm, w_ref, tw_ref, te_ref, ring_ref, sems):
    i = pl.program_id(0)
    n = pl.num_programs(0)
    p = jax.lax.rem(i, 2)

    @pl.when(i == 0)
    def _prologue():
        for c in range(_NSUB):
            pltpu.make_async_copy(
                x_hbm.at[pl.ds(c * _CHUNK, _CHUNK), :],
                ring_ref.at[c], sems.at[c]).start()

    for c in range(_NSUB):
        slot = p * _NSUB + c
        pltpu.make_async_copy(
            x_hbm.at[pl.ds(i * _BLOCK + c * _CHUNK, _CHUNK), :],
            ring_ref.at[slot], sems.at[slot]).wait()

        @pl.when(i < n - 1)
        def _refill():
            nslot = (1 - p) * _NSUB + c
            pltpu.make_async_copy(
                x_hbm.at[pl.ds((i + 1) * _BLOCK + c * _CHUNK, _CHUNK), :],
                ring_ref.at[nslot], sems.at[nslot]).start()

        xc = ring_ref[slot]
        logits = jax.lax.dot_general(
            w_ref[...], xc, (((1,), (1,)), ((), ())),
            preferred_element_type=jnp.float32)
        _topk_softmax(logits, tw_ref, te_ref, c)


def kernel(x, W):
    tokens = x.shape[0]
    grid = (tokens // _BLOCK,)
    tw, te = pl.pallas_call(
        _router_step,
        grid=grid,
        in_specs=[
            pl.BlockSpec(memory_space=pl.ANY),
            pl.BlockSpec((_EXPERTS, _HIDDEN), lambda i: (0, 0)),
        ],
        out_specs=[
            pl.BlockSpec((_BLOCK, _K), lambda i: (i, 0)),
            pl.BlockSpec((_BLOCK, _K), lambda i: (i, 0)),
        ],
        out_shape=[
            jax.ShapeDtypeStruct((tokens, _K), jnp.float32),
            jax.ShapeDtypeStruct((tokens, _K), jnp.int32),
        ],
        scratch_shapes=[
            pltpu.VMEM((_NSLOT, _CHUNK, _HIDDEN), jnp.float32),
            pltpu.SemaphoreType.DMA((_NSLOT,)),
        ],
    )(x, W)
    return tw, te


# speedup vs baseline: 1.2747x; 1.2747x over previous
import jax
import jax.numpy as jnp
from jax.experimental import pallas as pl
from jax.experimental.pallas import tpu as pltpu

_HIDDEN = 4096
_EXPERTS = 64
_K = 8
_CHUNK = 128
_NSUB = 8
_BLOCK = _CHUNK * _NSUB


def _issue_block_copies(x_hbm, ring_ref, sems, step, parity):
    for c in range(_NSUB):
        pltpu.make_async_copy(
            x_hbm.at[pl.ds(step * _BLOCK + c * _CHUNK, _CHUNK), :],
            ring_ref.at[parity, pl.ds(c * _CHUNK, _CHUNK), :],
            sems.at[parity, c]).start()


def _wait_block_copies(x_hbm, ring_ref, sems, step, parity):
    for c in range(_NSUB):
        pltpu.make_async_copy(
            x_hbm.at[pl.ds(step * _BLOCK + c * _CHUNK, _CHUNK), :],
            ring_ref.at[parity, pl.ds(c * _CHUNK, _CHUNK), :],
            sems.at[parity, c]).wait()


def _router_step(x_hbm, w_ref, tw_ref, te_ref, ring_ref, sems):
    i = pl.program_id(0)
    n = pl.num_programs(0)
    p = jax.lax.rem(i, 2)

    @pl.when(i == 0)
    def _prologue():
        _issue_block_copies(x_hbm, ring_ref, sems, i, p)

    _wait_block_copies(x_hbm, ring_ref, sems, i, p)

    @pl.when(i < n - 1)
    def _refill():
        _issue_block_copies(x_hbm, ring_ref, sems, i + 1, 1 - p)

    xb = ring_ref[p]
    logits = jax.lax.dot_general(
        w_ref[...], xb, (((1,), (1,)), ((), ())),
        preferred_element_type=jnp.float32)

    eidx = jax.lax.broadcasted_iota(jnp.int32, logits.shape, 0)
    l = logits
    vals = []
    idxs = []
    for _ in range(_K):
        m = jnp.max(l, axis=0, keepdims=True)
        idx = jnp.min(jnp.where(l == m, eidx, _EXPERTS),
                      axis=0, keepdims=True)
        vals.append(m)
        idxs.append(idx)
        l = jnp.where(eidx == idx, -jnp.inf, l)
    v = jnp.concatenate(vals, axis=0)
    e = jnp.exp(v - v[0:1])
    wts = e / jnp.sum(e, axis=0, keepdims=True)
    tw_ref[...] = wts.T
    te_ref[...] = jnp.concatenate(idxs, axis=0).T


def kernel(x, W):
    tokens = x.shape[0]
    grid = (tokens // _BLOCK,)
    tw, te = pl.pallas_call(
        _router_step,
        grid=grid,
        in_specs=[
            pl.BlockSpec(memory_space=pl.ANY),
            pl.BlockSpec((_EXPERTS, _HIDDEN), lambda i: (0, 0)),
        ],
        out_specs=[
            pl.BlockSpec((_BLOCK, _K), lambda i: (i, 0)),
            pl.BlockSpec((_BLOCK, _K), lambda i: (i, 0)),
        ],
        out_shape=[
            jax.ShapeDtypeStruct((tokens, _K), jnp.float32),
            jax.ShapeDtypeStruct((tokens, _K), jnp.int32),
        ],
        scratch_shapes=[
            pltpu.VMEM((2, _BLOCK, _HIDDEN), jnp.float32),
            pltpu.SemaphoreType.DMA((2, _NSUB)),
        ],
    )(x, W)
    return tw, te


# speedup vs baseline: 1.4770x; 1.1587x over previous
import jax
import jax.numpy as jnp
from jax.experimental import pallas as pl
from jax.experimental.pallas import tpu as pltpu

_HIDDEN = 4096
_EXPERTS = 64
_K = 8
_HALF = 512
_BLOCK = 2 * _HALF


def _router_block(xa_ref, xb_ref, w_ref, tw_ref, te_ref):
    w = w_ref[...]
    la = jax.lax.dot_general(
        w, xa_ref[0], (((1,), (1,)), ((), ())),
        preferred_element_type=jnp.float32)
    lb = jax.lax.dot_general(
        w, xb_ref[0], (((1,), (1,)), ((), ())),
        preferred_element_type=jnp.float32)
    logits = jnp.concatenate([la, lb], axis=1)
    eidx = jax.lax.broadcasted_iota(jnp.int32, logits.shape, 0)
    l = logits
    vals = []
    idxs = []
    for _ in range(_K):
        m = jnp.max(l, axis=0, keepdims=True)
        idx = jnp.min(jnp.where(l == m, eidx, _EXPERTS),
                      axis=0, keepdims=True)
        vals.append(m)
        idxs.append(idx)
        l = jnp.where(eidx == idx, -jnp.inf, l)
    v = jnp.concatenate(vals, axis=0)
    e = jnp.exp(v - v[0:1])
    wts = e / jnp.sum(e, axis=0, keepdims=True)
    tw_ref[...] = wts.T
    te_ref[...] = jnp.concatenate(idxs, axis=0).T


def kernel(x, W):
    tokens = x.shape[0]
    x3 = x.reshape(tokens // _HALF, _HALF, _HIDDEN)
    grid = (tokens // _BLOCK,)
    tw, te = pl.pallas_call(
        _router_block,
        grid=grid,
        in_specs=[
            pl.BlockSpec((1, _HALF, _HIDDEN), lambda i: (2 * i, 0, 0)),
            pl.BlockSpec((1, _HALF, _HIDDEN), lambda i: (2 * i + 1, 0, 0)),
            pl.BlockSpec((_EXPERTS, _HIDDEN), lambda i: (0, 0)),
        ],
        out_specs=[
            pl.BlockSpec((_BLOCK, _K), lambda i: (i, 0)),
            pl.BlockSpec((_BLOCK, _K), lambda i: (i, 0)),
        ],
        out_shape=[
            jax.ShapeDtypeStruct((tokens, _K), jnp.float32),
            jax.ShapeDtypeStruct((tokens, _K), jnp.int32),
        ],
    )(x3, x3, W)
    return tw, te
